# baseline (speedup 1.0000x reference)
import functools

import jax
import jax.numpy as jnp
from jax import lax
from jax.experimental import pallas as pl
from jax.experimental.pallas import tpu as pltpu
from jax.experimental.pallas import tpu_sc as plsc

VOCAB = 100000
EMB_DIM = 6

_NC = 2
_NS = 16
_NW = _NC * _NS
_LANES = 16

_PROJ_BLOCK = 10240


def _proj_body(t_ref, w_ref, b_ref, o_ref):
    acc = t_ref[0] * w_ref[0, 0]
    for d in range(1, EMB_DIM):
        acc += t_ref[d] * w_ref[0, d]
    o_ref[...] = acc + b_ref[0]


def _project_table(table, W, b):
    tableT = table.T
    return pl.pallas_call(
        _proj_body,
        in_specs=[
            pl.BlockSpec(memory_space=pltpu.VMEM),
            pl.BlockSpec(memory_space=pltpu.SMEM),
            pl.BlockSpec(memory_space=pltpu.SMEM),
        ],
        out_specs=pl.BlockSpec(memory_space=pltpu.VMEM),
        out_shape=jax.ShapeDtypeStruct((VOCAB,), jnp.float32),
        compiler_params=pltpu.CompilerParams(
            allow_input_fusion=[True, False, False],
        ),
    )(tableT, W, b)


def _make_sc_gather(n_idx):
    assert n_idx % (_NW * _LANES) == 0
    bpw = n_idx // _NW

    mesh = plsc.VectorSubcoreMesh(core_axis_name="c", subcore_axis_name="s")

    @functools.partial(
        pl.kernel,
        mesh=mesh,
        out_type=jax.ShapeDtypeStruct((n_idx,), jnp.float32),
        compiler_params=pltpu.CompilerParams(needs_layout_passes=False),
        scratch_types=[
            pltpu.VMEM((VOCAB,), jnp.float32),
            pltpu.VMEM((bpw,), jnp.int32),
            pltpu.VMEM((bpw,), jnp.float32),
            pltpu.SemaphoreType.DMA,
            pltpu.SemaphoreType.DMA,
        ],
    )
    def gather_kernel(pt_hbm, idx_hbm, out_hbm, pt_v, idx_v, out_v, sem_pt, sem_ix):
        wid = lax.axis_index("s") * _NC + lax.axis_index("c")
        base = wid * bpw
        cp_pt = pltpu.async_copy(pt_hbm, pt_v, sem_pt)
        cp_ix = pltpu.async_copy(idx_hbm.at[pl.ds(base, bpw)], idx_v, sem_ix)
        cp_ix.wait()
        cp_pt.wait()

        @plsc.parallel_loop(0, bpw, _LANES, unroll=8)
        def _(i):
            ids = idx_v[pl.ds(i, _LANES)]
            out_v[pl.ds(i, _LANES)] = plsc.load_gather(pt_v, [ids])

        pltpu.sync_copy(out_v, out_hbm.at[pl.ds(base, bpw)])

    return gather_kernel


def kernel(input, table, W, b):
    B, F = input.shape
    idx = input.reshape(-1).astype(jnp.int32)
    pt = _project_table(table, W, b)
    out_flat = _make_sc_gather(idx.shape[0])(pt, idx)
    return out_flat.reshape(B, F, 1)


# speedup vs baseline: 1.0020x; 1.0020x over previous
import functools

import jax
import jax.numpy as jnp
from jax import lax
from jax.experimental import pallas as pl
from jax.experimental.pallas import tpu as pltpu
from jax.experimental.pallas import tpu_sc as plsc

VOCAB = 100000
EMB_DIM = 6

_NC = 2
_NS = 16
_NW = _NC * _NS
_LANES = 16

_PROJ_BLOCK = 10240


def _proj_body(t_ref, w_ref, b_ref, o_ref):
    acc = t_ref[0] * w_ref[0, 0]
    for d in range(1, EMB_DIM):
        acc += t_ref[d] * w_ref[0, d]
    o_ref[...] = acc + b_ref[0]


def _project_table(table, W, b):
    tableT = table.T
    return pl.pallas_call(
        _proj_body,
        in_specs=[
            pl.BlockSpec(memory_space=pltpu.VMEM),
            pl.BlockSpec(memory_space=pltpu.SMEM),
            pl.BlockSpec(memory_space=pltpu.SMEM),
        ],
        out_specs=pl.BlockSpec(memory_space=pltpu.VMEM),
        out_shape=jax.ShapeDtypeStruct((VOCAB,), jnp.float32),
        compiler_params=pltpu.CompilerParams(
            allow_input_fusion=[True, False, False],
        ),
    )(tableT, W, b)


def _make_sc_gather(n_idx):
    assert n_idx % (_NW * _LANES) == 0
    bpw = n_idx // _NW

    mesh = plsc.VectorSubcoreMesh(core_axis_name="c", subcore_axis_name="s")

    @functools.partial(
        pl.kernel,
        mesh=mesh,
        out_type=jax.ShapeDtypeStruct((n_idx,), jnp.float32),
        compiler_params=pltpu.CompilerParams(needs_layout_passes=False),
        scratch_types=[
            pltpu.VMEM((VOCAB,), jnp.float32),
            pltpu.VMEM((bpw,), jnp.int32),
            pltpu.VMEM((bpw,), jnp.float32),
            pltpu.SemaphoreType.DMA,
            pltpu.SemaphoreType.DMA,
        ],
    )
    def gather_kernel(pt_hbm, idx_hbm, out_hbm, pt_v, idx_v, out_v, sem_pt, sem_ix):
        wid = lax.axis_index("s") * _NC + lax.axis_index("c")
        base = wid * bpw
        cp_pt = pltpu.async_copy(pt_hbm, pt_v, sem_pt)
        cp_ix = pltpu.async_copy(idx_hbm.at[pl.ds(base, bpw)], idx_v, sem_ix)
        cp_ix.wait()
        cp_pt.wait()

        @plsc.parallel_loop(0, bpw, _LANES, unroll=16)
        def _(i):
            ids = idx_v[pl.ds(i, _LANES)]
            out_v[pl.ds(i, _LANES)] = plsc.load_gather(pt_v, [ids])

        pltpu.sync_copy(out_v, out_hbm.at[pl.ds(base, bpw)])

    return gather_kernel


def kernel(input, table, W, b):
    B, F = input.shape
    idx = input.reshape(-1).astype(jnp.int32)
    pt = _project_table(table, W, b)
    out_flat = _make_sc_gather(idx.shape[0])(pt, idx)
    return out_flat.reshape(B, F, 1)
